# baseline (speedup 1.0000x reference)
import functools

import jax
import jax.numpy as jnp
from jax import lax
from jax.experimental import pallas as pl
from jax.experimental.pallas import tpu as pltpu
from jax.experimental.pallas import tpu_sc as plsc

_N = 10000
_E = 320000
_H = 128
_G = 64
_NT = 10

_NC = 2
_NS = 16
_NW = _NC * _NS
_CH = 128
_NCH = 80
_EPW = _CH * _NCH
_EPAD = _NW * _EPW
_NPAD = 10240
_BN = 1000


_sc_mesh = plsc.VectorSubcoreMesh(core_axis_name="c", subcore_axis_name="s")


@functools.partial(
    pl.kernel,
    mesh=_sc_mesh,
    out_type=jax.ShapeDtypeStruct((_NC, _NPAD, _H), jnp.float32),
    scratch_types=[
        pltpu.VMEM((_NCH, _CH), jnp.int32),
        pltpu.VMEM((_CH, _H), jnp.float32),
        pltpu.VMEM_SHARED((_NPAD, _H), jnp.float32),
    ],
)
def _sc_deg(dst2_hbm, ones_hbm, z_hbm, out_hbm, didx2, ones_v, acc):
    c = lax.axis_index("c")
    s = lax.axis_index("s")
    wid = c * _NS + s

    @pl.when(s == 0)
    def _():
        pltpu.sync_copy(z_hbm, acc)

    pltpu.sync_copy(dst2_hbm.at[wid], didx2)
    pltpu.sync_copy(ones_hbm, ones_v)
    plsc.subcore_barrier()

    def body(j, carry):
        pltpu.sync_copy(ones_v, acc.at[didx2.at[j]], add=True)
        return carry

    lax.fori_loop(0, _NCH, body, 0)
    plsc.subcore_barrier()

    @pl.when(s == 0)
    def _():
        pltpu.sync_copy(acc, out_hbm.at[c])


@functools.partial(
    pl.kernel,
    mesh=_sc_mesh,
    out_type=jax.ShapeDtypeStruct((_NC, _NPAD, _H), jnp.float32),
    scratch_types=[
        pltpu.VMEM((_NCH // 2, _CH), jnp.int32),
        pltpu.VMEM((_NCH // 2, _CH), jnp.int32),
        pltpu.VMEM((_CH, _H), jnp.float32),
        pltpu.VMEM((_CH, _H), jnp.float32),
        pltpu.VMEM_SHARED((_NPAD, _H), jnp.float32),
        pltpu.SemaphoreType.DMA,
        pltpu.SemaphoreType.DMA,
    ],
)
def _sc_edge(g_hbm, src2_hbm, dst2_hbm, z_hbm, out_hbm, sidx2, didx2, rows0,
             rows1, acc, sem0, sem1):
    c = lax.axis_index("c")
    s = lax.axis_index("s")
    wid = c * _NS + s

    @pl.when(s == 0)
    def _():
        pltpu.sync_copy(z_hbm, acc)

    plsc.subcore_barrier()

    for half in range(2):
        hb = half * (_NCH // 2)
        pltpu.sync_copy(src2_hbm.at[wid, pl.ds(hb, _NCH // 2)], sidx2)
        pltpu.sync_copy(dst2_hbm.at[wid, pl.ds(hb, _NCH // 2)], didx2)
        pltpu.async_copy(g_hbm.at[sidx2.at[0]], rows0, sem0)

        def body(t, carry):
            j0 = 2 * t
            j1 = j0 + 1
            pltpu.async_copy(g_hbm.at[sidx2.at[j1]], rows1, sem1)
            pltpu.make_async_copy(g_hbm.at[sidx2.at[j0]], rows0, sem0).wait()
            pltpu.sync_copy(rows0, acc.at[didx2.at[j0]], add=True)

            @pl.when(t < _NCH // 4 - 1)
            def _():
                pltpu.async_copy(g_hbm.at[sidx2.at[j0 + 2]], rows0, sem0)

            pltpu.make_async_copy(g_hbm.at[sidx2.at[j1]], rows1, sem1).wait()
            pltpu.sync_copy(rows1, acc.at[didx2.at[j1]], add=True)
            return carry

        lax.fori_loop(0, _NCH // 4, body, 0)
    plsc.subcore_barrier()

    @pl.when(s == 0)
    def _():
        pltpu.sync_copy(acc, out_hbm.at[c])


def _tc1_body(x_r, te_r, d2_r, w1_r, wt_r, bt_r, g1_o, te_o, dv_o):
    d2 = d2_r[...]
    deg = jnp.sum(d2[0] + d2[1], axis=1) * (1.0 / _H) + 1.0
    dinv = lax.rsqrt(deg)[:, None]
    g1_o[...] = jnp.dot(x_r[...], w1_r[...],
                        preferred_element_type=jnp.float32) * dinv
    te_o[...] = jnp.maximum(
        jnp.dot(te_r[...], wt_r[...], preferred_element_type=jnp.float32)
        + bt_r[...], 0.0)
    dv_o[...] = dinv


def _tc_mid_temb_body(s_r, g_r, dv_r, b_r, w_r, te_r, gn_o):
    sr = s_r[...]
    dv = dv_r[...]
    h = jnp.maximum((sr[0] + sr[1] + g_r[...]) * dv + b_r[...], 0.0) + te_r[...]
    gn_o[...] = jnp.dot(h, w_r[...], preferred_element_type=jnp.float32) * dv


def _tc_mid_body(s_r, g_r, dv_r, b_r, w_r, gn_o):
    sr = s_r[...]
    dv = dv_r[...]
    h = jnp.maximum((sr[0] + sr[1] + g_r[...]) * dv + b_r[...], 0.0)
    gn_o[...] = jnp.dot(h, w_r[...], preferred_element_type=jnp.float32) * dv


def _tc_pool_body(s_r, g_r, dv_r, b_r, ba_r, ms_o, mc_o):
    i = pl.program_id(0)
    sr = s_r[...]
    h = jnp.maximum((sr[0] + sr[1] + g_r[...]) * dv_r[...] + b_r[...], 0.0)
    bb = ba_r[0]
    gids = lax.broadcasted_iota(jnp.int32, (_G, _BN), 0)
    mask = (gids == bb).astype(jnp.float32)
    ps = jnp.dot(mask, h, preferred_element_type=jnp.float32)
    pc = jnp.broadcast_to(jnp.sum(mask, axis=1, keepdims=True), (_G, _H))

    @pl.when(i == 0)
    def _():
        ms_o[...] = ps
        mc_o[...] = pc

    @pl.when(i != 0)
    def _():
        ms_o[...] = ms_o[...] + ps
        mc_o[...] = mc_o[...] + pc


def _tc_out_body(ms_r, mc_r, wo_r, bo_r, o_r):
    pooled = ms_r[...] / jnp.maximum(mc_r[...], 1.0)
    o_r[...] = jnp.dot(pooled, wo_r[...],
                       preferred_element_type=jnp.float32) + bo_r[...]


def _row_spec(i):
    return (i, 0)


def kernel(x, edge_index, t_embedding, batch, Wt, bt, W1, b1, W2, b2, W3, b3,
           Wo, bo):
    npad_e = _EPAD - _E
    src2 = jnp.concatenate(
        [edge_index[0], jnp.zeros((npad_e,), jnp.int32)]).reshape(
            _NW, _NCH, _CH)
    dst2 = jnp.concatenate(
        [edge_index[1], jnp.full((npad_e,), _N, jnp.int32)]).reshape(
            _NW, _NCH, _CH)
    zacc = jnp.zeros((_NPAD, _H), jnp.float32)
    onesr = jnp.ones((_CH, _H), jnp.float32)

    deg2 = _sc_deg(dst2, onesr, zacc)

    grid = (_N // _BN,)
    row = pl.BlockSpec((_BN, _H), _row_spec)
    col1 = pl.BlockSpec((_BN, 1), _row_spec)
    wsp = pl.BlockSpec((_H, _H), lambda i: (0, 0))
    bsp = pl.BlockSpec((1, _H), lambda i: (0, 0))
    ssp = pl.BlockSpec((_NC, _BN, _H), lambda i: (0, i, 0))

    g1, temb, dinv = pl.pallas_call(
        _tc1_body,
        grid=grid,
        in_specs=[row, row, ssp, wsp, wsp, bsp],
        out_specs=[row, row, col1],
        out_shape=[jax.ShapeDtypeStruct((_N, _H), jnp.float32),
                   jax.ShapeDtypeStruct((_N, _H), jnp.float32),
                   jax.ShapeDtypeStruct((_N, 1), jnp.float32)],
    )(x, t_embedding, deg2, W1, Wt, bt.reshape(1, _H))

    s1 = _sc_edge(g1, src2, dst2, zacc)

    g2 = pl.pallas_call(
        _tc_mid_temb_body,
        grid=grid,
        in_specs=[ssp, row, col1, bsp, wsp, row],
        out_specs=row,
        out_shape=jax.ShapeDtypeStruct((_N, _H), jnp.float32),
    )(s1, g1, dinv, b1.reshape(1, _H), W2, temb)

    s2 = _sc_edge(g2, src2, dst2, zacc)

    g3 = pl.pallas_call(
        _tc_mid_body,
        grid=grid,
        in_specs=[ssp, row, col1, bsp, wsp],
        out_specs=row,
        out_shape=jax.ShapeDtypeStruct((_N, _H), jnp.float32),
    )(s2, g2, dinv, b2.reshape(1, _H), W3)

    s3 = _sc_edge(g3, src2, dst2, zacc)

    msum, mcnt = pl.pallas_call(
        _tc_pool_body,
        grid=grid,
        in_specs=[ssp, row, col1, bsp,
                  pl.BlockSpec((1, 1, _BN), lambda i: (i, 0, 0))],
        out_specs=[pl.BlockSpec((_G, _H), lambda i: (0, 0)),
                   pl.BlockSpec((_G, _H), lambda i: (0, 0))],
        out_shape=[jax.ShapeDtypeStruct((_G, _H), jnp.float32),
                   jax.ShapeDtypeStruct((_G, _H), jnp.float32)],
    )(s3, g3, dinv, b3.reshape(1, _H),
      batch.reshape(_N // _BN, 1, _BN))

    wo_pad = jnp.zeros((_H, _H), jnp.float32).at[:, :_NT].set(Wo)
    bo_pad = jnp.zeros((1, _H), jnp.float32).at[0, :_NT].set(bo)

    out = pl.pallas_call(
        _tc_out_body,
        grid=(1,),
        in_specs=[pl.BlockSpec((_G, _H), lambda i: (0, 0)),
                  pl.BlockSpec((_G, _H), lambda i: (0, 0)),
                  wsp, bsp],
        out_specs=pl.BlockSpec((_G, _H), lambda i: (0, 0)),
        out_shape=jax.ShapeDtypeStruct((_G, _H), jnp.float32),
    )(msum, mcnt, wo_pad, bo_pad)

    return out[:, :_NT]


# speedup vs baseline: 1.1845x; 1.1845x over previous
import functools

import jax
import jax.numpy as jnp
from jax import lax
from jax.experimental import pallas as pl
from jax.experimental.pallas import tpu as pltpu
from jax.experimental.pallas import tpu_sc as plsc

_N = 10000
_E = 320000
_H = 128
_G = 64
_NT = 10

_NC = 2
_NS = 16
_NW = _NC * _NS
_CH = 128
_NCH = 80
_EPW = _CH * _NCH
_EPAD = _NW * _EPW
_NPAD = 10240
_BN = 1000


_sc_mesh = plsc.VectorSubcoreMesh(core_axis_name="c", subcore_axis_name="s")


@functools.partial(
    pl.kernel,
    mesh=_sc_mesh,
    out_type=jax.ShapeDtypeStruct((_NC, _NPAD, _H), jnp.float32),
    scratch_types=[
        pltpu.VMEM((_NCH, _CH), jnp.int32),
        pltpu.VMEM((_CH, _H), jnp.float32),
        pltpu.VMEM_SHARED((_NPAD, _H), jnp.float32),
    ],
)
def _sc_deg(dst2_hbm, ones_hbm, z_hbm, out_hbm, didx2, ones_v, acc):
    c = lax.axis_index("c")
    s = lax.axis_index("s")
    wid = c * _NS + s

    @pl.when(s == 0)
    def _():
        pltpu.sync_copy(z_hbm, acc)

    pltpu.sync_copy(dst2_hbm.at[wid], didx2)
    pltpu.sync_copy(ones_hbm, ones_v)
    plsc.subcore_barrier()

    def body(j, carry):
        pltpu.sync_copy(ones_v, acc.at[didx2.at[j]], add=True)
        return carry

    lax.fori_loop(0, _NCH, body, 0)
    plsc.subcore_barrier()

    @pl.when(s == 0)
    def _():
        pltpu.sync_copy(acc, out_hbm.at[c])


@functools.partial(
    pl.kernel,
    mesh=_sc_mesh,
    out_type=jax.ShapeDtypeStruct((_NC, _NPAD, _H), jnp.float32),
    scratch_types=[
        pltpu.VMEM((_NCH // 2, _CH), jnp.int32),
        pltpu.VMEM((_NCH // 2, _CH), jnp.int32),
        pltpu.VMEM((_CH, _H), jnp.float32),
        pltpu.VMEM((_CH, _H), jnp.float32),
        pltpu.VMEM_SHARED((_NPAD, _H), jnp.float32),
        pltpu.SemaphoreType.DMA,
        pltpu.SemaphoreType.DMA,
    ],
)
def _sc_edge(g_hbm, src2_hbm, dst2_hbm, z_hbm, out_hbm, sidx2, didx2, rows0,
             rows1, acc, sem0, sem1):
    c = lax.axis_index("c")
    s = lax.axis_index("s")
    wid = c * _NS + s

    @pl.when(s == 0)
    def _():
        pltpu.sync_copy(z_hbm, acc)

    plsc.subcore_barrier()

    for half in range(2):
        hb = half * (_NCH // 2)
        pltpu.sync_copy(src2_hbm.at[wid, pl.ds(hb, _NCH // 2)], sidx2)
        pltpu.sync_copy(dst2_hbm.at[wid, pl.ds(hb, _NCH // 2)], didx2)
        pltpu.async_copy(g_hbm.at[sidx2.at[0]], rows0, sem0)

        def body(t, carry):
            j0 = 2 * t
            j1 = j0 + 1
            pltpu.async_copy(g_hbm.at[sidx2.at[j1]], rows1, sem1)
            pltpu.make_async_copy(g_hbm.at[sidx2.at[j0]], rows0, sem0).wait()
            pltpu.sync_copy(rows0, acc.at[didx2.at[j0]], add=True)

            @pl.when(t < _NCH // 4 - 1)
            def _():
                pltpu.async_copy(g_hbm.at[sidx2.at[j0 + 2]], rows0, sem0)

            pltpu.make_async_copy(g_hbm.at[sidx2.at[j1]], rows1, sem1).wait()
            pltpu.sync_copy(rows1, acc.at[didx2.at[j1]], add=True)
            return carry

        lax.fori_loop(0, _NCH // 4, body, 0)
    plsc.subcore_barrier()

    @pl.when(s == 0)
    def _():
        pltpu.sync_copy(acc, out_hbm.at[c])


def _tc1_body(x_r, te_r, d2_r, w1_r, wt_r, bt_r, g1_o, te_o, dv_o):
    d2 = d2_r[...]
    deg = jnp.sum(d2[0] + d2[1], axis=1) * (1.0 / _H) + 1.0
    dinv = lax.rsqrt(deg)[:, None]
    g1_o[...] = jnp.dot(x_r[...], w1_r[...],
                        preferred_element_type=jnp.float32) * dinv
    te_o[...] = jnp.maximum(
        jnp.dot(te_r[...], wt_r[...], preferred_element_type=jnp.float32)
        + bt_r[...], 0.0)
    dv_o[...] = dinv


def _tc_mid_temb_body(s_r, g_r, dv_r, b_r, w_r, te_r, gn_o):
    sr = s_r[...]
    dv = dv_r[...]
    h = jnp.maximum((sr[0] + sr[1] + g_r[...]) * dv + b_r[...], 0.0) + te_r[...]
    gn_o[...] = jnp.dot(h, w_r[...], preferred_element_type=jnp.float32) * dv


def _tc_mid_body(s_r, g_r, dv_r, b_r, w_r, gn_o):
    sr = s_r[...]
    dv = dv_r[...]
    h = jnp.maximum((sr[0] + sr[1] + g_r[...]) * dv + b_r[...], 0.0)
    gn_o[...] = jnp.dot(h, w_r[...], preferred_element_type=jnp.float32) * dv


def _tc_pool_body(s_r, g_r, dv_r, b_r, ba_r, ms_o, mc_o):
    i = pl.program_id(0)
    sr = s_r[...]
    h = jnp.maximum((sr[0] + sr[1] + g_r[...]) * dv_r[...] + b_r[...], 0.0)
    bb = ba_r[0]
    gids = lax.broadcasted_iota(jnp.int32, (_G, _BN), 0)
    mask = (gids == bb).astype(jnp.float32)
    ps = jnp.dot(mask, h, preferred_element_type=jnp.float32)
    pc = jnp.broadcast_to(jnp.sum(mask, axis=1, keepdims=True), (_G, _H))

    @pl.when(i == 0)
    def _():
        ms_o[...] = ps
        mc_o[...] = pc

    @pl.when(i != 0)
    def _():
        ms_o[...] = ms_o[...] + ps
        mc_o[...] = mc_o[...] + pc


def _tc_out_body(ms_r, mc_r, wo_r, bo_r, o_r):
    pooled = ms_r[...] / jnp.maximum(mc_r[...], 1.0)
    o_r[...] = jnp.dot(pooled, wo_r[...],
                       preferred_element_type=jnp.float32) + bo_r[...]


def _row_spec(i):
    return (i, 0)


def kernel(x, edge_index, t_embedding, batch, Wt, bt, W1, b1, W2, b2, W3, b3,
           Wo, bo):
    rpw = _E // _NW
    ppw = _EPW - rpw
    pad_s = jnp.zeros((_NW, ppw), jnp.int32)
    pad_d = jnp.broadcast_to(_N + jnp.arange(ppw, dtype=jnp.int32),
                             (_NW, ppw))
    src2 = jnp.concatenate(
        [edge_index[0].reshape(_NW, rpw), pad_s], axis=1).reshape(
            _NW, _NCH, _CH)
    dst2 = jnp.concatenate(
        [edge_index[1].reshape(_NW, rpw), pad_d], axis=1).reshape(
            _NW, _NCH, _CH)
    zacc = jnp.zeros((_NPAD, _H), jnp.float32)
    onesr = jnp.ones((_CH, _H), jnp.float32)

    deg2 = _sc_deg(dst2, onesr, zacc)

    grid = (_N // _BN,)
    row = pl.BlockSpec((_BN, _H), _row_spec)
    col1 = pl.BlockSpec((_BN, 1), _row_spec)
    wsp = pl.BlockSpec((_H, _H), lambda i: (0, 0))
    bsp = pl.BlockSpec((1, _H), lambda i: (0, 0))
    ssp = pl.BlockSpec((_NC, _BN, _H), lambda i: (0, i, 0))

    g1, temb, dinv = pl.pallas_call(
        _tc1_body,
        grid=grid,
        in_specs=[row, row, ssp, wsp, wsp, bsp],
        out_specs=[row, row, col1],
        out_shape=[jax.ShapeDtypeStruct((_N, _H), jnp.float32),
                   jax.ShapeDtypeStruct((_N, _H), jnp.float32),
                   jax.ShapeDtypeStruct((_N, 1), jnp.float32)],
    )(x, t_embedding, deg2, W1, Wt, bt.reshape(1, _H))

    s1 = _sc_edge(g1, src2, dst2, zacc)

    g2 = pl.pallas_call(
        _tc_mid_temb_body,
        grid=grid,
        in_specs=[ssp, row, col1, bsp, wsp, row],
        out_specs=row,
        out_shape=jax.ShapeDtypeStruct((_N, _H), jnp.float32),
    )(s1, g1, dinv, b1.reshape(1, _H), W2, temb)

    s2 = _sc_edge(g2, src2, dst2, zacc)

    g3 = pl.pallas_call(
        _tc_mid_body,
        grid=grid,
        in_specs=[ssp, row, col1, bsp, wsp],
        out_specs=row,
        out_shape=jax.ShapeDtypeStruct((_N, _H), jnp.float32),
    )(s2, g2, dinv, b2.reshape(1, _H), W3)

    s3 = _sc_edge(g3, src2, dst2, zacc)

    msum, mcnt = pl.pallas_call(
        _tc_pool_body,
        grid=grid,
        in_specs=[ssp, row, col1, bsp,
                  pl.BlockSpec((1, 1, _BN), lambda i: (i, 0, 0))],
        out_specs=[pl.BlockSpec((_G, _H), lambda i: (0, 0)),
                   pl.BlockSpec((_G, _H), lambda i: (0, 0))],
        out_shape=[jax.ShapeDtypeStruct((_G, _H), jnp.float32),
                   jax.ShapeDtypeStruct((_G, _H), jnp.float32)],
    )(s3, g3, dinv, b3.reshape(1, _H),
      batch.reshape(_N // _BN, 1, _BN))

    wo_pad = jnp.zeros((_H, _H), jnp.float32).at[:, :_NT].set(Wo)
    bo_pad = jnp.zeros((1, _H), jnp.float32).at[0, :_NT].set(bo)

    out = pl.pallas_call(
        _tc_out_body,
        grid=(1,),
        in_specs=[pl.BlockSpec((_G, _H), lambda i: (0, 0)),
                  pl.BlockSpec((_G, _H), lambda i: (0, 0)),
                  wsp, bsp],
        out_specs=pl.BlockSpec((_G, _H), lambda i: (0, 0)),
        out_shape=jax.ShapeDtypeStruct((_G, _H), jnp.float32),
    )(msum, mcnt, wo_pad, bo_pad)

    return out[:, :_NT]


# speedup vs baseline: 1.7185x; 1.4508x over previous
import functools

import jax
import jax.numpy as jnp
from jax import lax
from jax.experimental import pallas as pl
from jax.experimental.pallas import tpu as pltpu
from jax.experimental.pallas import tpu_sc as plsc

_N = 10000
_E = 320000
_H = 128
_G = 64
_NT = 10

_NC = 2
_NS = 16
_NW = _NC * _NS
_CH = 128
_NCH = 80
_EPW = _CH * _NCH
_EPAD = _NW * _EPW
_NPAD = 10240
_ECH = 80
_REPW = _E // _NW
_ENCH = _REPW // _ECH
_BN = 1000


_sc_mesh = plsc.VectorSubcoreMesh(core_axis_name="c", subcore_axis_name="s")


@functools.partial(
    pl.kernel,
    mesh=_sc_mesh,
    out_type=jax.ShapeDtypeStruct((_NC, _NPAD, _H), jnp.float32),
    scratch_types=[
        pltpu.VMEM((_NCH, _CH), jnp.int32),
        pltpu.VMEM((_CH, _H), jnp.float32),
        pltpu.VMEM_SHARED((_NPAD, _H), jnp.float32),
    ],
)
def _sc_deg(dst2_hbm, ones_hbm, z_hbm, out_hbm, didx2, ones_v, acc):
    c = lax.axis_index("c")
    s = lax.axis_index("s")
    wid = c * _NS + s

    @pl.when(s == 0)
    def _():
        pltpu.sync_copy(z_hbm, acc)

    pltpu.sync_copy(dst2_hbm.at[wid], didx2)
    pltpu.sync_copy(ones_hbm, ones_v)
    plsc.subcore_barrier()

    def body(j, carry):
        pltpu.sync_copy(ones_v, acc.at[didx2.at[j]], add=True)
        return carry

    lax.fori_loop(0, _NCH, body, 0)
    plsc.subcore_barrier()

    @pl.when(s == 0)
    def _():
        pltpu.sync_copy(acc, out_hbm.at[c])


@functools.partial(
    pl.kernel,
    mesh=_sc_mesh,
    out_type=jax.ShapeDtypeStruct((_NC, _NPAD, _H), jnp.float32),
    scratch_types=[
        pltpu.VMEM((_ECH,), jnp.int32),
        pltpu.VMEM((_ECH,), jnp.int32),
        pltpu.VMEM((_ECH, _H), jnp.float32),
        pltpu.VMEM_SHARED((_NPAD, _H), jnp.float32),
        pltpu.SemaphoreType.DMA,
    ],
)
def _sc_edge(g_hbm, src_hbm, dst_hbm, z_hbm, out_hbm, sidx, didx, rows, acc,
             sem):
    c = lax.axis_index("c")
    s = lax.axis_index("s")
    wid = c * _NS + s

    @pl.when(s == 0)
    def _():
        pltpu.sync_copy(z_hbm, acc)

    plsc.subcore_barrier()
    base = wid * _REPW

    def body(j, carry):
        e0 = base + j * _ECH
        pltpu.sync_copy(src_hbm.at[pl.ds(e0, _ECH)], sidx)
        cp = pltpu.async_copy(g_hbm.at[sidx], rows, sem)
        pltpu.sync_copy(dst_hbm.at[pl.ds(e0, _ECH)], didx)
        cp.wait()
        pltpu.sync_copy(rows, acc.at[didx], add=True)
        return carry

    lax.fori_loop(0, _ENCH, body, 0)
    plsc.subcore_barrier()

    @pl.when(s == 0)
    def _():
        pltpu.sync_copy(acc, out_hbm.at[c])


def _tc1_body(x_r, te_r, d2_r, w1_r, wt_r, bt_r, g1_o, te_o, dv_o):
    d2 = d2_r[...]
    deg = jnp.sum(d2[0] + d2[1], axis=1) * (1.0 / _H) + 1.0
    dinv = lax.rsqrt(deg)[:, None]
    g1_o[...] = jnp.dot(x_r[...], w1_r[...],
                        preferred_element_type=jnp.float32) * dinv
    te_o[...] = jnp.maximum(
        jnp.dot(te_r[...], wt_r[...], preferred_element_type=jnp.float32)
        + bt_r[...], 0.0)
    dv_o[...] = dinv


def _tc_mid_temb_body(s_r, g_r, dv_r, b_r, w_r, te_r, gn_o):
    sr = s_r[...]
    dv = dv_r[...]
    h = jnp.maximum((sr[0] + sr[1] + g_r[...]) * dv + b_r[...], 0.0) + te_r[...]
    gn_o[...] = jnp.dot(h, w_r[...], preferred_element_type=jnp.float32) * dv


def _tc_mid_body(s_r, g_r, dv_r, b_r, w_r, gn_o):
    sr = s_r[...]
    dv = dv_r[...]
    h = jnp.maximum((sr[0] + sr[1] + g_r[...]) * dv + b_r[...], 0.0)
    gn_o[...] = jnp.dot(h, w_r[...], preferred_element_type=jnp.float32) * dv


def _tc_pool_body(s_r, g_r, dv_r, b_r, ba_r, ms_o, mc_o):
    i = pl.program_id(0)
    sr = s_r[...]
    h = jnp.maximum((sr[0] + sr[1] + g_r[...]) * dv_r[...] + b_r[...], 0.0)
    bb = ba_r[0]
    gids = lax.broadcasted_iota(jnp.int32, (_G, _BN), 0)
    mask = (gids == bb).astype(jnp.float32)
    ps = jnp.dot(mask, h, preferred_element_type=jnp.float32)
    pc = jnp.broadcast_to(jnp.sum(mask, axis=1, keepdims=True), (_G, _H))

    @pl.when(i == 0)
    def _():
        ms_o[...] = ps
        mc_o[...] = pc

    @pl.when(i != 0)
    def _():
        ms_o[...] = ms_o[...] + ps
        mc_o[...] = mc_o[...] + pc


def _tc_out_body(ms_r, mc_r, wo_r, bo_r, o_r):
    pooled = ms_r[...] / jnp.maximum(mc_r[...], 1.0)
    o_r[...] = jnp.dot(pooled, wo_r[...],
                       preferred_element_type=jnp.float32) + bo_r[...]


def _row_spec(i):
    return (i, 0)


def kernel(x, edge_index, t_embedding, batch, Wt, bt, W1, b1, W2, b2, W3, b3,
           Wo, bo):
    src = edge_index[0]
    dst = edge_index[1]
    rpw = _E // _NW
    ppw = _EPW - rpw
    pad_s = jnp.zeros((_NW, ppw), jnp.int32)
    pad_d = jnp.broadcast_to(_N + jnp.arange(ppw, dtype=jnp.int32),
                             (_NW, ppw))
    src2 = jnp.concatenate(
        [edge_index[0].reshape(_NW, rpw), pad_s], axis=1).reshape(
            _NW, _NCH, _CH)
    dst2 = jnp.concatenate(
        [edge_index[1].reshape(_NW, rpw), pad_d], axis=1).reshape(
            _NW, _NCH, _CH)
    zacc = jnp.zeros((_NPAD, _H), jnp.float32)
    onesr = jnp.ones((_CH, _H), jnp.float32)

    deg2 = _sc_deg(dst2, onesr, zacc)

    grid = (_N // _BN,)
    row = pl.BlockSpec((_BN, _H), _row_spec)
    col1 = pl.BlockSpec((_BN, 1), _row_spec)
    wsp = pl.BlockSpec((_H, _H), lambda i: (0, 0))
    bsp = pl.BlockSpec((1, _H), lambda i: (0, 0))
    ssp = pl.BlockSpec((_NC, _BN, _H), lambda i: (0, i, 0))

    g1, temb, dinv = pl.pallas_call(
        _tc1_body,
        grid=grid,
        in_specs=[row, row, ssp, wsp, wsp, bsp],
        out_specs=[row, row, col1],
        out_shape=[jax.ShapeDtypeStruct((_N, _H), jnp.float32),
                   jax.ShapeDtypeStruct((_N, _H), jnp.float32),
                   jax.ShapeDtypeStruct((_N, 1), jnp.float32)],
    )(x, t_embedding, deg2, W1, Wt, bt.reshape(1, _H))

    s1 = _sc_edge(g1, src, dst, zacc)

    g2 = pl.pallas_call(
        _tc_mid_temb_body,
        grid=grid,
        in_specs=[ssp, row, col1, bsp, wsp, row],
        out_specs=row,
        out_shape=jax.ShapeDtypeStruct((_N, _H), jnp.float32),
    )(s1, g1, dinv, b1.reshape(1, _H), W2, temb)

    s2 = _sc_edge(g2, src, dst, zacc)

    g3 = pl.pallas_call(
        _tc_mid_body,
        grid=grid,
        in_specs=[ssp, row, col1, bsp, wsp],
        out_specs=row,
        out_shape=jax.ShapeDtypeStruct((_N, _H), jnp.float32),
    )(s2, g2, dinv, b2.reshape(1, _H), W3)

    s3 = _sc_edge(g3, src, dst, zacc)

    msum, mcnt = pl.pallas_call(
        _tc_pool_body,
        grid=grid,
        in_specs=[ssp, row, col1, bsp,
                  pl.BlockSpec((1, 1, _BN), lambda i: (i, 0, 0))],
        out_specs=[pl.BlockSpec((_G, _H), lambda i: (0, 0)),
                   pl.BlockSpec((_G, _H), lambda i: (0, 0))],
        out_shape=[jax.ShapeDtypeStruct((_G, _H), jnp.float32),
                   jax.ShapeDtypeStruct((_G, _H), jnp.float32)],
    )(s3, g3, dinv, b3.reshape(1, _H),
      batch.reshape(_N // _BN, 1, _BN))

    wo_pad = jnp.zeros((_H, _H), jnp.float32).at[:, :_NT].set(Wo)
    bo_pad = jnp.zeros((1, _H), jnp.float32).at[0, :_NT].set(bo)

    out = pl.pallas_call(
        _tc_out_body,
        grid=(1,),
        in_specs=[pl.BlockSpec((_G, _H), lambda i: (0, 0)),
                  pl.BlockSpec((_G, _H), lambda i: (0, 0)),
                  wsp, bsp],
        out_specs=pl.BlockSpec((_G, _H), lambda i: (0, 0)),
        out_shape=jax.ShapeDtypeStruct((_G, _H), jnp.float32),
    )(msum, mcnt, wo_pad, bo_pad)

    return out[:, :_NT]


# speedup vs baseline: 2.2728x; 1.3226x over previous
import functools

import jax
import jax.numpy as jnp
from jax import lax
from jax.experimental import pallas as pl
from jax.experimental.pallas import tpu as pltpu
from jax.experimental.pallas import tpu_sc as plsc

_N = 10000
_E = 320000
_H = 128
_G = 64
_NT = 10

_NC = 2
_NS = 16
_NW = _NC * _NS
_CH = 128
_NCH = 80
_EPW = _CH * _NCH
_EPAD = _NW * _EPW
_NPAD = 10240
_ECH = 80
_REPW = _E // _NW
_ENCH = _REPW // _ECH
_BN = 1000


_sc_mesh = plsc.VectorSubcoreMesh(core_axis_name="c", subcore_axis_name="s")


@functools.partial(
    pl.kernel,
    mesh=_sc_mesh,
    out_type=jax.ShapeDtypeStruct((_NC, _NPAD, _H), jnp.float32),
    scratch_types=[
        pltpu.VMEM((_NCH, _CH), jnp.int32),
        pltpu.VMEM((_CH, _H), jnp.float32),
        pltpu.VMEM_SHARED((_NPAD, _H), jnp.float32),
    ],
)
def _sc_deg(dst2_hbm, ones_hbm, z_hbm, out_hbm, didx2, ones_v, acc):
    c = lax.axis_index("c")
    s = lax.axis_index("s")
    wid = c * _NS + s

    @pl.when(s == 0)
    def _():
        pltpu.sync_copy(z_hbm, acc)

    pltpu.sync_copy(dst2_hbm.at[wid], didx2)
    pltpu.sync_copy(ones_hbm, ones_v)
    plsc.subcore_barrier()

    def body(j, carry):
        pltpu.sync_copy(ones_v, acc.at[didx2.at[j]], add=True)
        return carry

    lax.fori_loop(0, _NCH, body, 0)
    plsc.subcore_barrier()

    @pl.when(s == 0)
    def _():
        pltpu.sync_copy(acc, out_hbm.at[c])


@functools.partial(
    pl.kernel,
    mesh=_sc_mesh,
    out_type=jax.ShapeDtypeStruct((_NC, _NPAD, _H), jnp.float32),
    scratch_types=[
        pltpu.VMEM((_ECH,), jnp.int32),
        pltpu.VMEM((_ECH,), jnp.int32),
        pltpu.VMEM((_ECH,), jnp.int32),
        pltpu.VMEM((_ECH,), jnp.int32),
        pltpu.VMEM((_ECH, _H), jnp.float32),
        pltpu.VMEM((_ECH, _H), jnp.float32),
        pltpu.VMEM_SHARED((_NPAD, _H), jnp.float32),
        pltpu.SemaphoreType.DMA,
        pltpu.SemaphoreType.DMA,
    ],
)
def _sc_edge(g_hbm, src_hbm, dst_hbm, z_hbm, out_hbm, sidx0, sidx1, didx0,
             didx1, rows0, rows1, acc, sem0, sem1):
    c = lax.axis_index("c")
    s = lax.axis_index("s")
    wid = c * _NS + s

    @pl.when(s == 0)
    def _():
        pltpu.sync_copy(z_hbm, acc)

    plsc.subcore_barrier()
    base = wid * _REPW

    pltpu.sync_copy(src_hbm.at[pl.ds(base, _ECH)], sidx0)
    pltpu.async_copy(g_hbm.at[sidx0], rows0, sem0)

    def body(t, carry):
        e0 = base + 2 * t * _ECH
        pltpu.sync_copy(src_hbm.at[pl.ds(e0 + _ECH, _ECH)], sidx1)
        pltpu.async_copy(g_hbm.at[sidx1], rows1, sem1)
        pltpu.sync_copy(dst_hbm.at[pl.ds(e0, _ECH)], didx0)
        pltpu.make_async_copy(g_hbm.at[sidx0], rows0, sem0).wait()
        pltpu.sync_copy(rows0, acc.at[didx0], add=True)

        pltpu.sync_copy(src_hbm.at[pl.ds(e0 + 2 * _ECH, _ECH)], sidx0)
        pltpu.async_copy(g_hbm.at[sidx0], rows0, sem0)
        pltpu.sync_copy(dst_hbm.at[pl.ds(e0 + _ECH, _ECH)], didx1)
        pltpu.make_async_copy(g_hbm.at[sidx1], rows1, sem1).wait()
        pltpu.sync_copy(rows1, acc.at[didx1], add=True)
        return carry

    lax.fori_loop(0, (_ENCH - 1) // 2, body, 0)
    elast = base + (_ENCH - 1) * _ECH
    pltpu.sync_copy(dst_hbm.at[pl.ds(elast, _ECH)], didx0)
    pltpu.make_async_copy(g_hbm.at[sidx0], rows0, sem0).wait()
    pltpu.sync_copy(rows0, acc.at[didx0], add=True)
    plsc.subcore_barrier()

    @pl.when(s == 0)
    def _():
        pltpu.sync_copy(acc, out_hbm.at[c])


def _tc1_body(x_r, te_r, d2_r, w1_r, wt_r, bt_r, g1_o, te_o, dv_o):
    d2 = d2_r[...]
    deg = jnp.sum(d2[0] + d2[1], axis=1) * (1.0 / _H) + 1.0
    dinv = lax.rsqrt(deg)[:, None]
    g1_o[...] = jnp.dot(x_r[...], w1_r[...],
                        preferred_element_type=jnp.float32) * dinv
    te_o[...] = jnp.maximum(
        jnp.dot(te_r[...], wt_r[...], preferred_element_type=jnp.float32)
        + bt_r[...], 0.0)
    dv_o[...] = dinv


def _tc_mid_temb_body(s_r, g_r, dv_r, b_r, w_r, te_r, gn_o):
    sr = s_r[...]
    dv = dv_r[...]
    h = jnp.maximum((sr[0] + sr[1] + g_r[...]) * dv + b_r[...], 0.0) + te_r[...]
    gn_o[...] = jnp.dot(h, w_r[...], preferred_element_type=jnp.float32) * dv


def _tc_mid_body(s_r, g_r, dv_r, b_r, w_r, gn_o):
    sr = s_r[...]
    dv = dv_r[...]
    h = jnp.maximum((sr[0] + sr[1] + g_r[...]) * dv + b_r[...], 0.0)
    gn_o[...] = jnp.dot(h, w_r[...], preferred_element_type=jnp.float32) * dv


def _tc_pool_body(s_r, g_r, dv_r, b_r, ba_r, ms_o, mc_o):
    i = pl.program_id(0)
    sr = s_r[...]
    h = jnp.maximum((sr[0] + sr[1] + g_r[...]) * dv_r[...] + b_r[...], 0.0)
    bb = ba_r[0]
    gids = lax.broadcasted_iota(jnp.int32, (_G, _BN), 0)
    mask = (gids == bb).astype(jnp.float32)
    ps = jnp.dot(mask, h, preferred_element_type=jnp.float32)
    pc = jnp.broadcast_to(jnp.sum(mask, axis=1, keepdims=True), (_G, _H))

    @pl.when(i == 0)
    def _():
        ms_o[...] = ps
        mc_o[...] = pc

    @pl.when(i != 0)
    def _():
        ms_o[...] = ms_o[...] + ps
        mc_o[...] = mc_o[...] + pc


def _tc_out_body(ms_r, mc_r, wo_r, bo_r, o_r):
    pooled = ms_r[...] / jnp.maximum(mc_r[...], 1.0)
    o_r[...] = jnp.dot(pooled, wo_r[...],
                       preferred_element_type=jnp.float32) + bo_r[...]


def _row_spec(i):
    return (i, 0)


def kernel(x, edge_index, t_embedding, batch, Wt, bt, W1, b1, W2, b2, W3, b3,
           Wo, bo):
    src = edge_index[0]
    dst = edge_index[1]
    rpw = _E // _NW
    ppw = _EPW - rpw
    pad_s = jnp.zeros((_NW, ppw), jnp.int32)
    pad_d = jnp.broadcast_to(_N + jnp.arange(ppw, dtype=jnp.int32),
                             (_NW, ppw))
    src2 = jnp.concatenate(
        [edge_index[0].reshape(_NW, rpw), pad_s], axis=1).reshape(
            _NW, _NCH, _CH)
    dst2 = jnp.concatenate(
        [edge_index[1].reshape(_NW, rpw), pad_d], axis=1).reshape(
            _NW, _NCH, _CH)
    zacc = jnp.zeros((_NPAD, _H), jnp.float32)
    onesr = jnp.ones((_CH, _H), jnp.float32)

    deg2 = _sc_deg(dst2, onesr, zacc)

    grid = (_N // _BN,)
    row = pl.BlockSpec((_BN, _H), _row_spec)
    col1 = pl.BlockSpec((_BN, 1), _row_spec)
    wsp = pl.BlockSpec((_H, _H), lambda i: (0, 0))
    bsp = pl.BlockSpec((1, _H), lambda i: (0, 0))
    ssp = pl.BlockSpec((_NC, _BN, _H), lambda i: (0, i, 0))

    g1, temb, dinv = pl.pallas_call(
        _tc1_body,
        grid=grid,
        in_specs=[row, row, ssp, wsp, wsp, bsp],
        out_specs=[row, row, col1],
        out_shape=[jax.ShapeDtypeStruct((_N, _H), jnp.float32),
                   jax.ShapeDtypeStruct((_N, _H), jnp.float32),
                   jax.ShapeDtypeStruct((_N, 1), jnp.float32)],
    )(x, t_embedding, deg2, W1, Wt, bt.reshape(1, _H))

    s1 = _sc_edge(g1, src, dst, zacc)

    g2 = pl.pallas_call(
        _tc_mid_temb_body,
        grid=grid,
        in_specs=[ssp, row, col1, bsp, wsp, row],
        out_specs=row,
        out_shape=jax.ShapeDtypeStruct((_N, _H), jnp.float32),
    )(s1, g1, dinv, b1.reshape(1, _H), W2, temb)

    s2 = _sc_edge(g2, src, dst, zacc)

    g3 = pl.pallas_call(
        _tc_mid_body,
        grid=grid,
        in_specs=[ssp, row, col1, bsp, wsp],
        out_specs=row,
        out_shape=jax.ShapeDtypeStruct((_N, _H), jnp.float32),
    )(s2, g2, dinv, b2.reshape(1, _H), W3)

    s3 = _sc_edge(g3, src, dst, zacc)

    msum, mcnt = pl.pallas_call(
        _tc_pool_body,
        grid=grid,
        in_specs=[ssp, row, col1, bsp,
                  pl.BlockSpec((1, 1, _BN), lambda i: (i, 0, 0))],
        out_specs=[pl.BlockSpec((_G, _H), lambda i: (0, 0)),
                   pl.BlockSpec((_G, _H), lambda i: (0, 0))],
        out_shape=[jax.ShapeDtypeStruct((_G, _H), jnp.float32),
                   jax.ShapeDtypeStruct((_G, _H), jnp.float32)],
    )(s3, g3, dinv, b3.reshape(1, _H),
      batch.reshape(_N // _BN, 1, _BN))

    wo_pad = jnp.zeros((_H, _H), jnp.float32).at[:, :_NT].set(Wo)
    bo_pad = jnp.zeros((1, _H), jnp.float32).at[0, :_NT].set(bo)

    out = pl.pallas_call(
        _tc_out_body,
        grid=(1,),
        in_specs=[pl.BlockSpec((_G, _H), lambda i: (0, 0)),
                  pl.BlockSpec((_G, _H), lambda i: (0, 0)),
                  wsp, bsp],
        out_specs=pl.BlockSpec((_G, _H), lambda i: (0, 0)),
        out_shape=jax.ShapeDtypeStruct((_G, _H), jnp.float32),
    )(msum, mcnt, wo_pad, bo_pad)

    return out[:, :_NT]


# speedup vs baseline: 2.2748x; 1.0009x over previous
import functools

import jax
import jax.numpy as jnp
from jax import lax
from jax.experimental import pallas as pl
from jax.experimental.pallas import tpu as pltpu
from jax.experimental.pallas import tpu_sc as plsc

_N = 10000
_E = 320000
_H = 128
_G = 64
_NT = 10

_NC = 2
_NS = 16
_NW = _NC * _NS
_CH = 128
_NCH = 80
_EPW = _CH * _NCH
_EPAD = _NW * _EPW
_NPAD = 10240
_ECH = 80
_REPW = _E // _NW
_ENCH = _REPW // _ECH
_BN = 1000


_sc_mesh = plsc.VectorSubcoreMesh(core_axis_name="c", subcore_axis_name="s")


@functools.partial(
    pl.kernel,
    mesh=_sc_mesh,
    out_type=jax.ShapeDtypeStruct((_NC, _NPAD, _H), jnp.float32),
    scratch_types=[
        pltpu.VMEM((_NCH, _CH), jnp.int32),
        pltpu.VMEM((_CH, _H), jnp.float32),
        pltpu.VMEM_SHARED((_NPAD, _H), jnp.float32),
    ],
)
def _sc_deg(dst2_hbm, ones_hbm, z_hbm, out_hbm, didx2, ones_v, acc):
    c = lax.axis_index("c")
    s = lax.axis_index("s")
    wid = c * _NS + s

    @pl.when(s == 0)
    def _():
        pltpu.sync_copy(z_hbm, acc)

    pltpu.sync_copy(dst2_hbm.at[wid], didx2)
    pltpu.sync_copy(ones_hbm, ones_v)
    plsc.subcore_barrier()

    def body(j, carry):
        pltpu.sync_copy(ones_v, acc.at[didx2.at[j]], add=True)
        return carry

    lax.fori_loop(0, _NCH, body, 0)
    plsc.subcore_barrier()

    @pl.when(s == 0)
    def _():
        pltpu.sync_copy(acc, out_hbm.at[c])


@functools.partial(
    pl.kernel,
    mesh=_sc_mesh,
    out_type=jax.ShapeDtypeStruct((_NC, _NPAD, _H), jnp.float32),
    scratch_types=[
        [pltpu.VMEM((_ECH,), jnp.int32)] * 3,
        [pltpu.VMEM((_ECH,), jnp.int32)] * 3,
        [pltpu.VMEM((_ECH, _H), jnp.float32)] * 3,
        pltpu.VMEM_SHARED((_NPAD, _H), jnp.float32),
        [pltpu.SemaphoreType.DMA] * 3,
    ],
)
def _sc_edge(g_hbm, src_hbm, dst_hbm, z_hbm, out_hbm, sidx, didx, rows, acc,
             sem):
    c = lax.axis_index("c")
    s = lax.axis_index("s")
    wid = c * _NS + s

    @pl.when(s == 0)
    def _():
        pltpu.sync_copy(z_hbm, acc)

    plsc.subcore_barrier()
    base = wid * _REPW

    for b in range(3):
        pltpu.sync_copy(src_hbm.at[pl.ds(base + b * _ECH, _ECH)], sidx[b])
        pltpu.async_copy(g_hbm.at[sidx[b]], rows[b], sem[b])

    def body(t, carry):
        for b in range(3):
            j = 3 * t + b
            e0 = base + j * _ECH
            pltpu.sync_copy(dst_hbm.at[pl.ds(e0, _ECH)], didx[b])
            pltpu.make_async_copy(g_hbm.at[sidx[b]], rows[b], sem[b]).wait()
            pltpu.sync_copy(rows[b], acc.at[didx[b]], add=True)

            @pl.when(j + 3 < _ENCH)
            def _():
                pltpu.sync_copy(src_hbm.at[pl.ds(e0 + 3 * _ECH, _ECH)],
                                sidx[b])
                pltpu.async_copy(g_hbm.at[sidx[b]], rows[b], sem[b])
        return carry

    lax.fori_loop(0, _ENCH // 3, body, 0)
    for b in range(_ENCH - 3 * (_ENCH // 3)):
        e0 = base + (3 * (_ENCH // 3) + b) * _ECH
        pltpu.sync_copy(dst_hbm.at[pl.ds(e0, _ECH)], didx[b])
        pltpu.make_async_copy(g_hbm.at[sidx[b]], rows[b], sem[b]).wait()
        pltpu.sync_copy(rows[b], acc.at[didx[b]], add=True)
    plsc.subcore_barrier()

    @pl.when(s == 0)
    def _():
        pltpu.sync_copy(acc, out_hbm.at[c])


def _tc1_body(x_r, te_r, d2_r, w1_r, wt_r, bt_r, g1_o, te_o, dv_o):
    d2 = d2_r[...]
    deg = jnp.sum(d2[0] + d2[1], axis=1) * (1.0 / _H) + 1.0
    dinv = lax.rsqrt(deg)[:, None]
    g1_o[...] = jnp.dot(x_r[...], w1_r[...],
                        preferred_element_type=jnp.float32) * dinv
    te_o[...] = jnp.maximum(
        jnp.dot(te_r[...], wt_r[...], preferred_element_type=jnp.float32)
        + bt_r[...], 0.0)
    dv_o[...] = dinv


def _tc_mid_temb_body(s_r, g_r, dv_r, b_r, w_r, te_r, gn_o):
    sr = s_r[...]
    dv = dv_r[...]
    h = jnp.maximum((sr[0] + sr[1] + g_r[...]) * dv + b_r[...], 0.0) + te_r[...]
    gn_o[...] = jnp.dot(h, w_r[...], preferred_element_type=jnp.float32) * dv


def _tc_mid_body(s_r, g_r, dv_r, b_r, w_r, gn_o):
    sr = s_r[...]
    dv = dv_r[...]
    h = jnp.maximum((sr[0] + sr[1] + g_r[...]) * dv + b_r[...], 0.0)
    gn_o[...] = jnp.dot(h, w_r[...], preferred_element_type=jnp.float32) * dv


def _tc_pool_body(s_r, g_r, dv_r, b_r, ba_r, ms_o, mc_o):
    i = pl.program_id(0)
    sr = s_r[...]
    h = jnp.maximum((sr[0] + sr[1] + g_r[...]) * dv_r[...] + b_r[...], 0.0)
    bb = ba_r[0]
    gids = lax.broadcasted_iota(jnp.int32, (_G, _BN), 0)
    mask = (gids == bb).astype(jnp.float32)
    ps = jnp.dot(mask, h, preferred_element_type=jnp.float32)
    pc = jnp.broadcast_to(jnp.sum(mask, axis=1, keepdims=True), (_G, _H))

    @pl.when(i == 0)
    def _():
        ms_o[...] = ps
        mc_o[...] = pc

    @pl.when(i != 0)
    def _():
        ms_o[...] = ms_o[...] + ps
        mc_o[...] = mc_o[...] + pc


def _tc_out_body(ms_r, mc_r, wo_r, bo_r, o_r):
    pooled = ms_r[...] / jnp.maximum(mc_r[...], 1.0)
    o_r[...] = jnp.dot(pooled, wo_r[...],
                       preferred_element_type=jnp.float32) + bo_r[...]


def _row_spec(i):
    return (i, 0)


def kernel(x, edge_index, t_embedding, batch, Wt, bt, W1, b1, W2, b2, W3, b3,
           Wo, bo):
    src = edge_index[0]
    dst = edge_index[1]
    rpw = _E // _NW
    ppw = _EPW - rpw
    pad_s = jnp.zeros((_NW, ppw), jnp.int32)
    pad_d = jnp.broadcast_to(_N + jnp.arange(ppw, dtype=jnp.int32),
                             (_NW, ppw))
    src2 = jnp.concatenate(
        [edge_index[0].reshape(_NW, rpw), pad_s], axis=1).reshape(
            _NW, _NCH, _CH)
    dst2 = jnp.concatenate(
        [edge_index[1].reshape(_NW, rpw), pad_d], axis=1).reshape(
            _NW, _NCH, _CH)
    zacc = jnp.zeros((_NPAD, _H), jnp.float32)
    onesr = jnp.ones((_CH, _H), jnp.float32)

    deg2 = _sc_deg(dst2, onesr, zacc)

    grid = (_N // _BN,)
    row = pl.BlockSpec((_BN, _H), _row_spec)
    col1 = pl.BlockSpec((_BN, 1), _row_spec)
    wsp = pl.BlockSpec((_H, _H), lambda i: (0, 0))
    bsp = pl.BlockSpec((1, _H), lambda i: (0, 0))
    ssp = pl.BlockSpec((_NC, _BN, _H), lambda i: (0, i, 0))

    g1, temb, dinv = pl.pallas_call(
        _tc1_body,
        grid=grid,
        in_specs=[row, row, ssp, wsp, wsp, bsp],
        out_specs=[row, row, col1],
        out_shape=[jax.ShapeDtypeStruct((_N, _H), jnp.float32),
                   jax.ShapeDtypeStruct((_N, _H), jnp.float32),
                   jax.ShapeDtypeStruct((_N, 1), jnp.float32)],
    )(x, t_embedding, deg2, W1, Wt, bt.reshape(1, _H))

    s1 = _sc_edge(g1, src, dst, zacc)

    g2 = pl.pallas_call(
        _tc_mid_temb_body,
        grid=grid,
        in_specs=[ssp, row, col1, bsp, wsp, row],
        out_specs=row,
        out_shape=jax.ShapeDtypeStruct((_N, _H), jnp.float32),
    )(s1, g1, dinv, b1.reshape(1, _H), W2, temb)

    s2 = _sc_edge(g2, src, dst, zacc)

    g3 = pl.pallas_call(
        _tc_mid_body,
        grid=grid,
        in_specs=[ssp, row, col1, bsp, wsp],
        out_specs=row,
        out_shape=jax.ShapeDtypeStruct((_N, _H), jnp.float32),
    )(s2, g2, dinv, b2.reshape(1, _H), W3)

    s3 = _sc_edge(g3, src, dst, zacc)

    msum, mcnt = pl.pallas_call(
        _tc_pool_body,
        grid=grid,
        in_specs=[ssp, row, col1, bsp,
                  pl.BlockSpec((1, 1, _BN), lambda i: (i, 0, 0))],
        out_specs=[pl.BlockSpec((_G, _H), lambda i: (0, 0)),
                   pl.BlockSpec((_G, _H), lambda i: (0, 0))],
        out_shape=[jax.ShapeDtypeStruct((_G, _H), jnp.float32),
                   jax.ShapeDtypeStruct((_G, _H), jnp.float32)],
    )(s3, g3, dinv, b3.reshape(1, _H),
      batch.reshape(_N // _BN, 1, _BN))

    wo_pad = jnp.zeros((_H, _H), jnp.float32).at[:, :_NT].set(Wo)
    bo_pad = jnp.zeros((1, _H), jnp.float32).at[0, :_NT].set(bo)

    out = pl.pallas_call(
        _tc_out_body,
        grid=(1,),
        in_specs=[pl.BlockSpec((_G, _H), lambda i: (0, 0)),
                  pl.BlockSpec((_G, _H), lambda i: (0, 0)),
                  wsp, bsp],
        out_specs=pl.BlockSpec((_G, _H), lambda i: (0, 0)),
        out_shape=jax.ShapeDtypeStruct((_G, _H), jnp.float32),
    )(msum, mcnt, wo_pad, bo_pad)

    return out[:, :_NT]


# speedup vs baseline: 2.2749x; 1.0000x over previous
import functools

import jax
import jax.numpy as jnp
from jax import lax
from jax.experimental import pallas as pl
from jax.experimental.pallas import tpu as pltpu
from jax.experimental.pallas import tpu_sc as plsc

_N = 10000
_E = 320000
_H = 128
_G = 64
_NT = 10

_NC = 2
_NS = 16
_NW = _NC * _NS
_CH = 128
_NCH = 80
_EPW = _CH * _NCH
_EPAD = _NW * _EPW
_NPAD = 10240
_ECH = 80
_REPW = _E // _NW
_ENCH = _REPW // _ECH
_BN = 1000


_sc_mesh = plsc.VectorSubcoreMesh(core_axis_name="c", subcore_axis_name="s")


@functools.partial(
    pl.kernel,
    mesh=_sc_mesh,
    out_type=jax.ShapeDtypeStruct((_NC, _NPAD, _H), jnp.float32),
    scratch_types=[
        pltpu.VMEM((_NCH, _CH), jnp.int32),
        pltpu.VMEM((_CH, _H), jnp.float32),
        pltpu.VMEM_SHARED((_NPAD, _H), jnp.float32),
    ],
)
def _sc_deg(dst2_hbm, ones_hbm, z_hbm, out_hbm, didx2, ones_v, acc):
    c = lax.axis_index("c")
    s = lax.axis_index("s")
    wid = c * _NS + s

    @pl.when(s == 0)
    def _():
        pltpu.sync_copy(z_hbm, acc)

    pltpu.sync_copy(dst2_hbm.at[wid], didx2)
    pltpu.sync_copy(ones_hbm, ones_v)
    plsc.subcore_barrier()

    def body(j, carry):
        pltpu.sync_copy(ones_v, acc.at[didx2.at[j]], add=True)
        return carry

    lax.fori_loop(0, _NCH, body, 0)
    plsc.subcore_barrier()

    @pl.when(s == 0)
    def _():
        pltpu.sync_copy(acc, out_hbm.at[c])


@functools.partial(
    pl.kernel,
    mesh=_sc_mesh,
    out_type=jax.ShapeDtypeStruct((_NC, _NPAD, _H), jnp.float32),
    scratch_types=[
        [pltpu.VMEM((_ECH,), jnp.int32)] * 3,
        [pltpu.VMEM((_ECH,), jnp.int32)] * 3,
        [pltpu.VMEM((_ECH, _H), jnp.float32)] * 3,
        pltpu.VMEM_SHARED((_NPAD, _H), jnp.float32),
        [pltpu.SemaphoreType.DMA] * 3,
    ],
)
def _sc_edge(g_hbm, src_hbm, dst_hbm, z_hbm, out_hbm, sidx, didx, rows, acc,
             sem):
    c = lax.axis_index("c")
    s = lax.axis_index("s")
    wid = c * _NS + s

    @pl.when(s == 0)
    def _():
        pltpu.sync_copy(z_hbm, acc)

    plsc.subcore_barrier()
    base = wid * _REPW

    for b in range(3):
        pltpu.sync_copy(src_hbm.at[pl.ds(base + b * _ECH, _ECH)], sidx[b])
        pltpu.async_copy(g_hbm.at[sidx[b]], rows[b], sem[b])

    def body(t, carry):
        for b in range(3):
            j = 3 * t + b
            e0 = base + j * _ECH
            pltpu.sync_copy(dst_hbm.at[pl.ds(e0, _ECH)], didx[b])
            pltpu.make_async_copy(g_hbm.at[sidx[b]], rows[b], sem[b]).wait()
            pltpu.sync_copy(rows[b], acc.at[didx[b]], add=True)

            @pl.when(j + 3 < _ENCH)
            def _():
                pltpu.sync_copy(src_hbm.at[pl.ds(e0 + 3 * _ECH, _ECH)],
                                sidx[b])
                pltpu.async_copy(g_hbm.at[sidx[b]], rows[b], sem[b])
        return carry

    lax.fori_loop(0, _ENCH // 3, body, 0)
    for b in range(_ENCH - 3 * (_ENCH // 3)):
        e0 = base + (3 * (_ENCH // 3) + b) * _ECH
        pltpu.sync_copy(dst_hbm.at[pl.ds(e0, _ECH)], didx[b])
        pltpu.make_async_copy(g_hbm.at[sidx[b]], rows[b], sem[b]).wait()
        pltpu.sync_copy(rows[b], acc.at[didx[b]], add=True)
    plsc.subcore_barrier()

    @pl.when(s == 0)
    def _():
        pltpu.sync_copy(acc, out_hbm.at[c])


def _tc1_body(x_r, te_r, d2_r, w1_r, wt_r, bt_r, g1_o, te_o, dv_o):
    d2 = d2_r[...]
    deg = jnp.sum(d2[0] + d2[1], axis=1) * (1.0 / _H) + 1.0
    dinv = lax.rsqrt(deg)[:, None]
    g1_o[...] = jnp.dot(x_r[...], w1_r[...],
                        preferred_element_type=jnp.float32) * dinv
    te_o[...] = jnp.maximum(
        jnp.dot(te_r[...], wt_r[...], preferred_element_type=jnp.float32)
        + bt_r[...], 0.0)
    dv_o[...] = dinv


def _tc_mid_temb_body(s_r, g_r, dv_r, b_r, w_r, te_r, gn_o):
    sr = s_r[...]
    dv = dv_r[...]
    h = jnp.maximum((sr[0] + sr[1] + g_r[...]) * dv + b_r[...], 0.0) + te_r[...]
    gn_o[...] = jnp.dot(h, w_r[...], preferred_element_type=jnp.float32) * dv


def _tc_mid_body(s_r, g_r, dv_r, b_r, w_r, gn_o):
    sr = s_r[...]
    dv = dv_r[...]
    h = jnp.maximum((sr[0] + sr[1] + g_r[...]) * dv + b_r[...], 0.0)
    gn_o[...] = jnp.dot(h, w_r[...], preferred_element_type=jnp.float32) * dv


def _tc_pool_body(s_r, g_r, dv_r, b_r, ba_r, ms_o, mc_o):
    i = pl.program_id(0)
    sr = s_r[...]
    h = jnp.maximum((sr[0] + sr[1] + g_r[...]) * dv_r[...] + b_r[...], 0.0)
    bb = ba_r[0]
    gids = lax.broadcasted_iota(jnp.int32, (_G, _BN), 0)
    mask = (gids == bb).astype(jnp.float32)
    ps = jnp.dot(mask, h, preferred_element_type=jnp.float32)
    pc = jnp.broadcast_to(jnp.sum(mask, axis=1, keepdims=True), (_G, _H))

    @pl.when(i == 0)
    def _():
        ms_o[...] = ps
        mc_o[...] = pc

    @pl.when(i != 0)
    def _():
        ms_o[...] = ms_o[...] + ps
        mc_o[...] = mc_o[...] + pc


def _tc_out_body(ms_r, mc_r, wo_r, bo_r, o_r):
    pooled = ms_r[...] / jnp.maximum(mc_r[...], 1.0)
    o_r[...] = jnp.dot(pooled, wo_r[...],
                       preferred_element_type=jnp.float32) + bo_r[...]


def _row_spec(i):
    return (i, 0)


def kernel(x, edge_index, t_embedding, batch, Wt, bt, W1, b1, W2, b2, W3, b3,
           Wo, bo):
    src = edge_index[0]
    dst = edge_index[1]
    rpw = _E // _NW
    ppw = _EPW - rpw
    pad_d = jnp.broadcast_to(_N + jnp.arange(ppw, dtype=jnp.int32),
                             (_NW, ppw))
    dst2 = jnp.concatenate(
        [dst.reshape(_NW, rpw), pad_d], axis=1).reshape(_NW, _NCH, _CH)
    zacc = jnp.zeros((_NPAD, _H), jnp.float32)
    onesr = jnp.ones((_CH, _H), jnp.float32)

    deg2 = _sc_deg(dst2, onesr, zacc)

    grid = (_N // _BN,)
    row = pl.BlockSpec((_BN, _H), _row_spec)
    col1 = pl.BlockSpec((_BN, 1), _row_spec)
    wsp = pl.BlockSpec((_H, _H), lambda i: (0, 0))
    bsp = pl.BlockSpec((1, _H), lambda i: (0, 0))
    ssp = pl.BlockSpec((_NC, _BN, _H), lambda i: (0, i, 0))

    g1, temb, dinv = pl.pallas_call(
        _tc1_body,
        grid=grid,
        in_specs=[row, row, ssp, wsp, wsp, bsp],
        out_specs=[row, row, col1],
        out_shape=[jax.ShapeDtypeStruct((_N, _H), jnp.float32),
                   jax.ShapeDtypeStruct((_N, _H), jnp.float32),
                   jax.ShapeDtypeStruct((_N, 1), jnp.float32)],
    )(x, t_embedding, deg2, W1, Wt, bt.reshape(1, _H))

    s1 = _sc_edge(g1, src, dst, zacc)

    g2 = pl.pallas_call(
        _tc_mid_temb_body,
        grid=grid,
        in_specs=[ssp, row, col1, bsp, wsp, row],
        out_specs=row,
        out_shape=jax.ShapeDtypeStruct((_N, _H), jnp.float32),
    )(s1, g1, dinv, b1.reshape(1, _H), W2, temb)

    s2 = _sc_edge(g2, src, dst, zacc)

    g3 = pl.pallas_call(
        _tc_mid_body,
        grid=grid,
        in_specs=[ssp, row, col1, bsp, wsp],
        out_specs=row,
        out_shape=jax.ShapeDtypeStruct((_N, _H), jnp.float32),
    )(s2, g2, dinv, b2.reshape(1, _H), W3)

    s3 = _sc_edge(g3, src, dst, zacc)

    msum, mcnt = pl.pallas_call(
        _tc_pool_body,
        grid=grid,
        in_specs=[ssp, row, col1, bsp,
                  pl.BlockSpec((1, 1, _BN), lambda i: (i, 0, 0))],
        out_specs=[pl.BlockSpec((_G, _H), lambda i: (0, 0)),
                   pl.BlockSpec((_G, _H), lambda i: (0, 0))],
        out_shape=[jax.ShapeDtypeStruct((_G, _H), jnp.float32),
                   jax.ShapeDtypeStruct((_G, _H), jnp.float32)],
    )(s3, g3, dinv, b3.reshape(1, _H),
      batch.reshape(_N // _BN, 1, _BN))

    wo_pad = jnp.zeros((_H, _H), jnp.float32).at[:, :_NT].set(Wo)
    bo_pad = jnp.zeros((1, _H), jnp.float32).at[0, :_NT].set(bo)

    out = pl.pallas_call(
        _tc_out_body,
        grid=(1,),
        in_specs=[pl.BlockSpec((_G, _H), lambda i: (0, 0)),
                  pl.BlockSpec((_G, _H), lambda i: (0, 0)),
                  wsp, bsp],
        out_specs=pl.BlockSpec((_G, _H), lambda i: (0, 0)),
        out_shape=jax.ShapeDtypeStruct((_G, _H), jnp.float32),
    )(msum, mcnt, wo_pad, bo_pad)

    return out[:, :_NT]
